# Initial kernel scaffold; baseline (speedup 1.0000x reference)
import jax
import jax.numpy as jnp
from jax.experimental import pallas as pl


def kernel(input_, embed_weight, linear_weight):
    raise NotImplementedError("write your pallas kernel here")


# speedup vs baseline: 2.1338x; 2.1338x over previous
import functools

import jax
import jax.numpy as jnp
from jax import lax
from jax.experimental import pallas as pl
from jax.experimental.pallas import tpu as pltpu
from jax.experimental.pallas import tpu_sc as plsc


def _embedding_bag_sc(idx2d, table, hist, bags_per_chunk):
    info = plsc.get_sparse_core_info()
    nc, ns, lanes = info.num_cores, info.num_subcores, info.num_lanes
    nw = nc * ns
    n_chunks_total, chunk_idx = idx2d.shape
    assert chunk_idx == bags_per_chunk * hist
    v, d = table.shape
    n_bags = n_chunks_total * bags_per_chunk
    assert n_bags % nw == 0
    bags_pw = n_bags // nw
    chunks_pw = n_chunks_total // nw
    n_col = d // lanes

    mesh = plsc.VectorSubcoreMesh(core_axis_name="c", subcore_axis_name="s")

    @functools.partial(
        pl.kernel,
        out_type=jax.ShapeDtypeStruct((n_bags, d), jnp.float32),
        mesh=mesh,
        scratch_types=[
            pltpu.VMEM((chunks_pw, chunk_idx), jnp.int32),
            pltpu.VMEM((chunk_idx, d), jnp.float32),
            pltpu.VMEM((bags_pw, d), jnp.float32),
            pltpu.SemaphoreType.DMA,
        ],
        compiler_params=pltpu.CompilerParams(use_tc_tiling_on_sc=False),
    )
    def k(idx_hbm, table_hbm, out_hbm, idx_v, rows_v, pooled_v, sem):
        wid = lax.axis_index("s") * nc + lax.axis_index("c")
        pltpu.sync_copy(idx_hbm.at[pl.ds(wid * chunks_pw, chunks_pw), :], idx_v)

        def chunk_body(ci, _):
            pltpu.async_copy(table_hbm.at[idx_v.at[ci]], rows_v, sem).wait()
            for b in range(bags_per_chunk):
                def row_body(r, accs):
                    base = b * hist + r
                    return tuple(
                        accs[c] + rows_v[base, pl.ds(c * lanes, lanes)]
                        for c in range(n_col)
                    )
                accs = tuple(
                    jnp.zeros((lanes,), jnp.float32) for _ in range(n_col)
                )
                accs = lax.fori_loop(0, hist, row_body, accs)
                bag = ci * bags_per_chunk + b
                for c in range(n_col):
                    pooled_v[bag, pl.ds(c * lanes, lanes)] = accs[c]
            return 0

        lax.fori_loop(0, chunks_pw, chunk_body, 0)
        pltpu.sync_copy(pooled_v, out_hbm.at[pl.ds(wid * bags_pw, bags_pw), :])

    return k(idx2d, table)


def _proj_tc(pooled, w, block_b=2048):
    b, d = pooled.shape
    o, _ = w.shape

    def body(p_ref, w_ref, o_ref):
        o_ref[...] = lax.dot_general(
            p_ref[...], w_ref[...],
            (((1,), (1,)), ((), ())),
            preferred_element_type=jnp.float32,
        )

    return pl.pallas_call(
        body,
        grid=(b // block_b,),
        in_specs=[
            pl.BlockSpec((block_b, d), lambda i: (i, 0)),
            pl.BlockSpec((o, d), lambda i: (0, 0)),
        ],
        out_specs=pl.BlockSpec((block_b, o), lambda i: (i, 0)),
        out_shape=jax.ShapeDtypeStruct((b, o), jnp.float32),
    )(pooled, w)


def kernel(input_, embed_weight, linear_weight):
    batch, hist = input_.shape
    bags_per_chunk = 2
    chunk_idx = bags_per_chunk * hist
    idx2d = input_.reshape(batch // bags_per_chunk, chunk_idx).astype(jnp.int32)
    pooled = _embedding_bag_sc(idx2d, embed_weight, hist, bags_per_chunk)
    return _proj_tc(pooled, linear_weight)


# speedup vs baseline: 2.2117x; 1.0365x over previous
import functools

import jax
import jax.numpy as jnp
from jax import lax
from jax.experimental import pallas as pl
from jax.experimental.pallas import tpu as pltpu
from jax.experimental.pallas import tpu_sc as plsc


def _project_table_tc(table, w, block_rows=8000):
    v, d = table.shape
    o, _ = w.shape
    assert v % block_rows == 0

    def body(t_ref, w_ref, o_ref):
        o_ref[...] = lax.dot_general(
            t_ref[...], w_ref[...],
            (((1,), (1,)), ((), ())),
            preferred_element_type=jnp.float32,
        )

    return pl.pallas_call(
        body,
        grid=(v // block_rows,),
        in_specs=[
            pl.BlockSpec((block_rows, d), lambda i: (i, 0)),
            pl.BlockSpec((o, d), lambda i: (0, 0)),
        ],
        out_specs=pl.BlockSpec((block_rows, o), lambda i: (i, 0)),
        out_shape=jax.ShapeDtypeStruct((v, o), jnp.float32),
    )(table, w)


def _bag_sum_sc(idx2d, ptable, hist, bags_per_chunk):
    info = plsc.get_sparse_core_info()
    nc, ns, lanes = info.num_cores, info.num_subcores, info.num_lanes
    nw = nc * ns
    n_chunks_total, chunk_idx = idx2d.shape
    assert chunk_idx == bags_per_chunk * hist
    _, o = ptable.shape
    n_bags = n_chunks_total * bags_per_chunk
    assert n_bags % nw == 0
    bags_pw = n_bags // nw
    chunks_pw = n_chunks_total // nw
    assert chunks_pw % 2 == 0
    n_col = o // lanes

    mesh = plsc.VectorSubcoreMesh(core_axis_name="c", subcore_axis_name="s")

    @functools.partial(
        pl.kernel,
        out_type=jax.ShapeDtypeStruct((n_bags, o), jnp.float32),
        mesh=mesh,
        scratch_types=[
            pltpu.VMEM((chunks_pw, chunk_idx), jnp.int32),
            pltpu.VMEM((2, chunk_idx, o), jnp.float32),
            pltpu.VMEM((bags_pw, o), jnp.float32),
            pltpu.SemaphoreType.DMA,
            pltpu.SemaphoreType.DMA,
        ],
    )
    def k(idx_hbm, ptable_hbm, out_hbm, idx_v, rows_v, pooled_v, sem0, sem1):
        wid = lax.axis_index("s") * nc + lax.axis_index("c")
        pltpu.sync_copy(idx_hbm.at[pl.ds(wid * chunks_pw, chunks_pw), :], idx_v)

        def start(ci, buf, sem):
            pltpu.async_copy(ptable_hbm.at[idx_v.at[ci]], rows_v.at[buf], sem)

        def wait(buf, sem):
            pltpu.make_async_copy(
                ptable_hbm.at[idx_v.at[0]], rows_v.at[buf], sem
            ).wait()

        def compute(ci, buf):
            for b in range(bags_per_chunk):
                def row_body(r, accs):
                    base = b * hist + r
                    return tuple(
                        accs[c] + rows_v[buf, base, pl.ds(c * lanes, lanes)]
                        for c in range(n_col)
                    )
                accs = tuple(
                    jnp.zeros((lanes,), jnp.float32) for _ in range(n_col)
                )
                accs = lax.fori_loop(0, hist, row_body, accs)
                bag = ci * bags_per_chunk + b
                for c in range(n_col):
                    pooled_v[bag, pl.ds(c * lanes, lanes)] = accs[c]

        start(0, 0, sem0)

        def pair_body(ci2, _):
            ci = ci2 * 2
            start(ci + 1, 1, sem1)
            wait(0, sem0)
            compute(ci, 0)

            @pl.when(ci2 + 1 < chunks_pw // 2)
            def _():
                start(ci + 2, 0, sem0)

            wait(1, sem1)
            compute(ci + 1, 1)
            return 0

        lax.fori_loop(0, chunks_pw // 2, pair_body, 0)
        pltpu.sync_copy(pooled_v, out_hbm.at[pl.ds(wid * bags_pw, bags_pw), :])

    return k(idx2d, ptable)


def kernel(input_, embed_weight, linear_weight):
    batch, hist = input_.shape
    bags_per_chunk = 2
    chunk_idx = bags_per_chunk * hist
    idx2d = input_.reshape(batch // bags_per_chunk, chunk_idx).astype(jnp.int32)
    ptable = _project_table_tc(embed_weight, linear_weight)
    return _bag_sum_sc(idx2d, ptable, hist, bags_per_chunk)


# speedup vs baseline: 3.8271x; 1.7303x over previous
import functools

import jax
import jax.numpy as jnp
from jax import lax
from jax.experimental import pallas as pl
from jax.experimental.pallas import tpu as pltpu
from jax.experimental.pallas import tpu_sc as plsc


def _project_table_tc(table_t, w, block_rows=8000):
    d, v = table_t.shape
    o, _ = w.shape

    def body(t_ref, w_ref, o_ref):
        o_ref[...] = lax.dot_general(
            t_ref[...], w_ref[...],
            (((0,), (1,)), ((), ())),
            preferred_element_type=jnp.float32,
        )

    return pl.pallas_call(
        body,
        grid=((v + block_rows - 1) // block_rows,),
        in_specs=[
            pl.BlockSpec((d, block_rows), lambda i: (0, i)),
            pl.BlockSpec((o, d), lambda i: (0, 0)),
        ],
        out_specs=pl.BlockSpec((block_rows, o), lambda i: (i, 0)),
        out_shape=jax.ShapeDtypeStruct((v, o), jnp.float32),
    )(table_t, w)


def _bag_sum_sc(idx2d, ptable, hist, bags_per_chunk):
    info = plsc.get_sparse_core_info()
    nc, ns, lanes = info.num_cores, info.num_subcores, info.num_lanes
    nw = nc * ns
    n_chunks_total, chunk_idx = idx2d.shape
    assert chunk_idx == bags_per_chunk * hist
    _, o = ptable.shape
    n_bags = n_chunks_total * bags_per_chunk
    assert n_bags % nw == 0
    bags_pw = n_bags // nw
    chunks_pw = n_chunks_total // nw
    assert chunks_pw % 2 == 0
    n_col = o // lanes

    mesh = plsc.VectorSubcoreMesh(core_axis_name="c", subcore_axis_name="s")

    @functools.partial(
        pl.kernel,
        out_type=jax.ShapeDtypeStruct((n_bags, o), jnp.float32),
        mesh=mesh,
        scratch_types=[
            pltpu.VMEM((chunks_pw, chunk_idx), jnp.int32),
            pltpu.VMEM((2, chunk_idx, o), jnp.float32),
            pltpu.VMEM((bags_pw, o), jnp.float32),
            pltpu.SemaphoreType.DMA,
            pltpu.SemaphoreType.DMA,
        ],
    )
    def k(idx_hbm, ptable_hbm, out_hbm, idx_v, rows_v, pooled_v, sem0, sem1):
        wid = lax.axis_index("s") * nc + lax.axis_index("c")
        pltpu.sync_copy(idx_hbm.at[pl.ds(wid * chunks_pw, chunks_pw), :], idx_v)

        def start(ci, buf, sem):
            pltpu.async_copy(ptable_hbm.at[idx_v.at[ci]], rows_v.at[buf], sem)

        def wait(buf, sem):
            pltpu.make_async_copy(
                ptable_hbm.at[idx_v.at[0]], rows_v.at[buf], sem
            ).wait()

        def compute(ci, buf):
            for b in range(bags_per_chunk):
                def row_body(r, accs):
                    base = b * hist + r
                    return tuple(
                        accs[c] + rows_v[buf, base, pl.ds(c * lanes, lanes)]
                        for c in range(n_col)
                    )
                accs = tuple(
                    jnp.zeros((lanes,), jnp.float32) for _ in range(n_col)
                )
                accs = lax.fori_loop(0, hist, row_body, accs)
                bag = ci * bags_per_chunk + b
                for c in range(n_col):
                    pooled_v[bag, pl.ds(c * lanes, lanes)] = accs[c]

        start(0, 0, sem0)

        def pair_body(ci2, _):
            ci = ci2 * 2
            start(ci + 1, 1, sem1)
            wait(0, sem0)
            compute(ci, 0)

            @pl.when(ci2 + 1 < chunks_pw // 2)
            def _():
                start(ci + 2, 0, sem0)

            wait(1, sem1)
            compute(ci + 1, 1)
            return 0

        lax.fori_loop(0, chunks_pw // 2, pair_body, 0)
        pltpu.sync_copy(pooled_v, out_hbm.at[pl.ds(wid * bags_pw, bags_pw), :])

    return k(idx2d, ptable)


def kernel(input_, embed_weight, linear_weight):
    batch, hist = input_.shape
    bags_per_chunk = 2
    chunk_idx = bags_per_chunk * hist
    idx2d = input_.reshape(batch // bags_per_chunk, chunk_idx).astype(jnp.int32)
    ptable = _project_table_tc(embed_weight.T, linear_weight, block_rows=8192)
    return _bag_sum_sc(idx2d, ptable, hist, bags_per_chunk)
